# Initial kernel scaffold; baseline (speedup 1.0000x reference)
import jax
import jax.numpy as jnp
from jax.experimental import pallas as pl


def kernel(x_nchw, wb1, b1, wb2, b2, wf1, bf1, wf2, bf2, wf3, bf3):
    raise NotImplementedError("write your pallas kernel here")


# speedup vs baseline: 2.3724x; 2.3724x over previous
import functools

import jax
import jax.numpy as jnp
from jax.experimental import pallas as pl
from jax.experimental.pallas import tpu as pltpu


def _lenet_body(x_ref, wb1_ref, b1_ref, wb2_ref, b2_ref,
                wf1_ref, bf1_ref, wf2_ref, bf2_ref,
                wf3_ref, bf3_ref, out_ref, *, tb):
    f32 = jnp.float32
    bf16 = jnp.bfloat16
    R1 = 30 * tb
    R2 = 25 * tb

    x = x_ref[...]
    acc1 = jnp.dot(x[0:R1], wb1_ref[0], preferred_element_type=f32)
    acc1 = acc1 + jnp.dot(x[tb:tb + R1], wb1_ref[1], preferred_element_type=f32)
    acc1 = acc1 + jnp.dot(x[2 * tb:2 * tb + R1], wb1_ref[2],
                          preferred_element_type=f32)
    y1 = jnp.maximum(acc1 + b1_ref[...], 0.0).astype(bf16)

    hm1 = jnp.maximum(y1[:-tb], y1[tb:])
    wm1 = jnp.maximum(hm1[:, :174], hm1[:, 6:180])

    acc2 = jnp.dot(wm1[0:R2], wb2_ref[0], preferred_element_type=f32)
    acc2 = acc2 + jnp.dot(wm1[2 * tb:2 * tb + R2], wb2_ref[1],
                          preferred_element_type=f32)
    acc2 = acc2 + jnp.dot(wm1[4 * tb:4 * tb + R2], wb2_ref[2],
                          preferred_element_type=f32)
    y2 = jnp.maximum(acc2 + b2_ref[...], 0.0).astype(bf16)

    hm2 = jnp.maximum(y2[:-2 * tb], y2[2 * tb:])
    wm2 = jnp.maximum(hm2[:, :192], hm2[:, 16:208])

    acc_f = jnp.dot(wm2[0:tb], wf1_ref[0], preferred_element_type=f32)
    for h4 in range(1, 6):
        acc_f = acc_f + jnp.dot(wm2[4 * h4 * tb:4 * h4 * tb + tb], wf1_ref[h4],
                                preferred_element_type=f32)
    z1 = jnp.maximum(acc_f + bf1_ref[...], 0.0).astype(bf16)

    z2 = jnp.maximum(jnp.dot(z1, wf2_ref[...], preferred_element_type=f32)
                     + bf2_ref[...], 0.0).astype(bf16)
    z3 = jnp.dot(z2, wf3_ref[...], preferred_element_type=f32) + bf3_ref[...]

    out_ref[...] = z3.astype(out_ref.dtype)


def kernel(x_nchw, wb1, b1, wb2, b2, wf1, bf1, wf2, bf2, wf3, bf3):
    f32 = jnp.float32
    bf16 = jnp.bfloat16
    tb = 128

    B = x_nchw.shape[0]
    Bp = ((B + tb - 1) // tb) * tb
    if Bp != B:
        x_nchw = jnp.pad(x_nchw, ((0, Bp - B), (0, 0), (0, 0), (0, 0)))
    G = Bp // tb

    xr = jnp.transpose(x_nchw.astype(bf16), (0, 2, 3, 1)).reshape(Bp, 32, 96)
    x_rows = jnp.transpose(xr.reshape(G, tb, 32, 96),
                           (0, 2, 1, 3)).reshape(G * 32 * tb, 96)

    body = functools.partial(_lenet_body, tb=tb)
    out = pl.pallas_call(
        body,
        out_shape=jax.ShapeDtypeStruct((Bp, 128), f32),
        grid=(G,),
        in_specs=[
            pl.BlockSpec((32 * tb, 96), lambda i: (i, 0)),
            pl.BlockSpec((3, 96, 180), lambda i: (0, 0, 0)),
            pl.BlockSpec((1, 180), lambda i: (0, 0)),
            pl.BlockSpec((3, 174, 208), lambda i: (0, 0, 0)),
            pl.BlockSpec((1, 208), lambda i: (0, 0)),
            pl.BlockSpec((6, 192, 120), lambda i: (0, 0, 0)),
            pl.BlockSpec((1, 120), lambda i: (0, 0)),
            pl.BlockSpec((120, 84), lambda i: (0, 0)),
            pl.BlockSpec((1, 84), lambda i: (0, 0)),
            pl.BlockSpec((84, 128), lambda i: (0, 0)),
            pl.BlockSpec((1, 128), lambda i: (0, 0)),
        ],
        out_specs=pl.BlockSpec((tb, 128), lambda i: (i, 0)),
        compiler_params=pltpu.CompilerParams(
            dimension_semantics=("parallel",),
            vmem_limit_bytes=64 * 1024 * 1024),
    )(x_rows, wb1.astype(bf16), b1, wb2.astype(bf16), b2,
      wf1.astype(bf16), bf1, wf2.astype(bf16), bf2, wf3.astype(bf16), bf3)

    return out[:B, :10]


# speedup vs baseline: 2.5152x; 1.0602x over previous
import functools

import jax
import jax.numpy as jnp
from jax.experimental import pallas as pl
from jax.experimental.pallas import tpu as pltpu


def _lenet_body(x_ref, wb1_ref, b1_ref, wb2_ref, b2_ref,
                wf1_ref, bf1_ref, wf2_ref, bf2_ref,
                wf3_ref, bf3_ref, out_ref, *, tb):
    f32 = jnp.float32
    bf16 = jnp.bfloat16
    R1 = 30 * tb
    R2 = 25 * tb

    xt = jnp.transpose(x_ref[...], (1, 0, 2))
    x = jnp.concatenate(
        [xt[32 * c:32 * (c + 1)].reshape(32 * tb, 32) for c in range(3)],
        axis=1).astype(bf16)
    acc1 = jnp.dot(x[0:R1], wb1_ref[0], preferred_element_type=f32)
    acc1 = acc1 + jnp.dot(x[tb:tb + R1], wb1_ref[1], preferred_element_type=f32)
    acc1 = acc1 + jnp.dot(x[2 * tb:2 * tb + R1], wb1_ref[2],
                          preferred_element_type=f32)
    y1 = jnp.maximum(acc1 + b1_ref[...], 0.0).astype(bf16)

    hm1 = jnp.maximum(y1[:-tb], y1[tb:])
    wm1 = jnp.maximum(hm1[:, :174], hm1[:, 6:180])

    acc2 = jnp.dot(wm1[0:R2], wb2_ref[0], preferred_element_type=f32)
    acc2 = acc2 + jnp.dot(wm1[2 * tb:2 * tb + R2], wb2_ref[1],
                          preferred_element_type=f32)
    acc2 = acc2 + jnp.dot(wm1[4 * tb:4 * tb + R2], wb2_ref[2],
                          preferred_element_type=f32)
    y2 = jnp.maximum(acc2 + b2_ref[...], 0.0).astype(bf16)

    hm2 = jnp.maximum(y2[:-2 * tb], y2[2 * tb:])
    wm2 = jnp.maximum(hm2[:, :192], hm2[:, 16:208])

    acc_f = jnp.dot(wm2[0:tb], wf1_ref[0], preferred_element_type=f32)
    for h4 in range(1, 6):
        acc_f = acc_f + jnp.dot(wm2[4 * h4 * tb:4 * h4 * tb + tb], wf1_ref[h4],
                                preferred_element_type=f32)
    z1 = jnp.maximum(acc_f + bf1_ref[...], 0.0).astype(bf16)

    z2 = jnp.maximum(jnp.dot(z1, wf2_ref[...], preferred_element_type=f32)
                     + bf2_ref[...], 0.0).astype(bf16)
    z3 = jnp.dot(z2, wf3_ref[...], preferred_element_type=f32) + bf3_ref[...]

    out_ref[...] = z3.astype(out_ref.dtype)


def kernel(x_nchw, wb1, b1, wb2, b2, wf1, bf1, wf2, bf2, wf3, bf3):
    f32 = jnp.float32
    bf16 = jnp.bfloat16
    tb = 128

    B = x_nchw.shape[0]
    Bp = ((B + tb - 1) // tb) * tb
    if Bp != B:
        x_nchw = jnp.pad(x_nchw, ((0, Bp - B), (0, 0), (0, 0), (0, 0)))
    G = Bp // tb

    x_rows = x_nchw.reshape(Bp, 96, 32)

    perm = jnp.arange(96)
    wi, ci = perm % 32, perm // 32
    wb1p = wb1[:, wi * 3 + ci, :]

    body = functools.partial(_lenet_body, tb=tb)
    out = pl.pallas_call(
        body,
        out_shape=jax.ShapeDtypeStruct((Bp, 128), f32),
        grid=(G,),
        in_specs=[
            pl.BlockSpec((tb, 96, 32), lambda i: (i, 0, 0)),
            pl.BlockSpec((3, 96, 180), lambda i: (0, 0, 0)),
            pl.BlockSpec((1, 180), lambda i: (0, 0)),
            pl.BlockSpec((3, 174, 208), lambda i: (0, 0, 0)),
            pl.BlockSpec((1, 208), lambda i: (0, 0)),
            pl.BlockSpec((6, 192, 120), lambda i: (0, 0, 0)),
            pl.BlockSpec((1, 120), lambda i: (0, 0)),
            pl.BlockSpec((120, 84), lambda i: (0, 0)),
            pl.BlockSpec((1, 84), lambda i: (0, 0)),
            pl.BlockSpec((84, 128), lambda i: (0, 0)),
            pl.BlockSpec((1, 128), lambda i: (0, 0)),
        ],
        out_specs=pl.BlockSpec((tb, 128), lambda i: (i, 0)),
        compiler_params=pltpu.CompilerParams(
            dimension_semantics=("parallel",),
            vmem_limit_bytes=64 * 1024 * 1024),
    )(x_rows, wb1p.astype(bf16), b1, wb2.astype(bf16), b2,
      wf1.astype(bf16), bf1, wf2.astype(bf16), bf2, wf3.astype(bf16), bf3)

    return out[:B, :10]
